# Initial kernel scaffold; baseline (speedup 1.0000x reference)
import jax
import jax.numpy as jnp
from jax.experimental import pallas as pl


def kernel(u, i, adj_entity, adj_relation, entity_table, relation_table, W1_w, W1_b, W2_w, W2_b, wl1_w, wl1_b, wl2_w, wl2_b, wl3_w, wl3_b):
    raise NotImplementedError("write your pallas kernel here")


# speedup vs baseline: 1.8064x; 1.8064x over previous
import functools

import jax
import jax.numpy as jnp
from jax import lax
from jax.experimental import pallas as pl
from jax.experimental.pallas import tpu as pltpu, tpu_sc as plsc


def _make_sc_ids(B, K, D, NC, NS):
    NW = NC * NS
    bw = B // NW
    NG = bw // 16
    mesh = plsc.VectorSubcoreMesh(core_axis_name="c", subcore_axis_name="s")

    @functools.partial(
        pl.kernel,
        mesh=mesh,
        out_type=[
            jax.ShapeDtypeStruct((B, 128), jnp.int32),
            jax.ShapeDtypeStruct((B, 128), jnp.int32),
            jax.ShapeDtypeStruct((B, D), jnp.float32),
            jax.ShapeDtypeStruct((B, D), jnp.float32),
        ],
        scratch_types=[
            pltpu.VMEM((bw,), jnp.int32),
            pltpu.VMEM((bw,), jnp.int32),
            pltpu.VMEM((bw,), jnp.int32),
            pltpu.VMEM((bw, 128), jnp.int32),
            pltpu.VMEM((bw, 128), jnp.int32),
            pltpu.VMEM((bw, D), jnp.float32),
            pltpu.VMEM((bw, D), jnp.float32),
            pltpu.SemaphoreType.DMA,
            pltpu.SemaphoreType.DMA,
            pltpu.SemaphoreType.DMA,
            pltpu.SemaphoreType.DMA,
        ],
    )
    def sc_ids(u_hbm, i_hbm, adj_ec_hbm, adj_rc_hbm, ent_hbm,
               aec_out, arc_out, h_out, u_out,
               i_v, u_v, ic_v, aec, arc, h_v, uu_v, s0, s1, s2, s3):
        wid = lax.axis_index("s") * NC + lax.axis_index("c")
        base = wid * bw
        pltpu.sync_copy(i_hbm.at[pl.ds(base, bw)], i_v)
        pltpu.sync_copy(u_hbm.at[pl.ds(base, bw)], u_v)
        for g in range(NG):
            ic_v[pl.ds(g * 16, 16)] = lax.shift_right_logical(
                i_v[pl.ds(g * 16, 16)], 3)
        c0 = pltpu.async_copy(adj_ec_hbm.at[ic_v], aec, s0)
        c1 = pltpu.async_copy(adj_rc_hbm.at[ic_v], arc, s1)
        c2 = pltpu.async_copy(ent_hbm.at[i_v], h_v, s2)
        c3 = pltpu.async_copy(ent_hbm.at[u_v], uu_v, s3)
        c0.wait()
        pltpu.sync_copy(aec, aec_out.at[pl.ds(base, bw)])
        c1.wait()
        pltpu.sync_copy(arc, arc_out.at[pl.ds(base, bw)])
        c2.wait()
        pltpu.sync_copy(h_v, h_out.at[pl.ds(base, bw)])
        c3.wait()
        pltpu.sync_copy(uu_v, u_out.at[pl.ds(base, bw)])

    return sc_ids


def _extract_body(K, i_ref, aec_ref, arc_ref, e_out, r_out):
    sel = i_ref[...] & 7
    aec = aec_ref[...]
    arc = arc_ref[...]
    e = jnp.zeros(e_out.shape, jnp.int32)
    r = jnp.zeros(r_out.shape, jnp.int32)
    for o in range(8):
        m = sel == o
        e = jnp.where(m, aec[:, o * K:(o + 1) * K], e)
        r = jnp.where(m, arc[:, o * K:(o + 1) * K], r)
    e_out[...] = e
    r_out[...] = r


def _tc_extract(i, aec, arc):
    B = i.shape[0]
    K = 16
    bb = 2048
    out = pl.pallas_call(
        functools.partial(_extract_body, K),
        grid=(B // bb,),
        in_specs=[
            pl.BlockSpec((bb, 1), lambda b: (b, 0)),
            pl.BlockSpec((bb, 128), lambda b: (b, 0)),
            pl.BlockSpec((bb, 128), lambda b: (b, 0)),
        ],
        out_specs=[
            pl.BlockSpec((bb, K), lambda b: (b, 0)),
            pl.BlockSpec((bb, K), lambda b: (b, 0)),
        ],
        out_shape=[
            jax.ShapeDtypeStruct((B, K), jnp.int32),
            jax.ShapeDtypeStruct((B, K), jnp.int32),
        ],
    )(i.reshape(B, 1), aec, arc)
    return out


def _make_sc_gather(M, D, NC, NS):
    NW = NC * NS
    rw = M // NW
    CH = 128
    nch = rw // CH
    mesh = plsc.VectorSubcoreMesh(core_axis_name="c", subcore_axis_name="s")

    @functools.partial(
        pl.kernel,
        mesh=mesh,
        out_type=jax.ShapeDtypeStruct((M, D), jnp.float32),
        scratch_types=[
            pltpu.VMEM((rw,), jnp.int32),
            pltpu.VMEM((CH, D), jnp.float32),
            pltpu.VMEM((CH, D), jnp.float32),
            pltpu.SemaphoreType.DMA,
            pltpu.SemaphoreType.DMA,
        ],
    )
    def sc_gather(idx_hbm, ent_hbm, out_hbm, idx_v, buf0, buf1, g0, g1):
        wid = lax.axis_index("s") * NC + lax.axis_index("c")
        base = wid * rw
        pltpu.sync_copy(idx_hbm.at[pl.ds(base, rw)], idx_v)
        bufs = (buf0, buf1)
        sems = (g0, g1)
        prev = None
        for c in range(nch):
            b = c % 2
            d = pltpu.async_copy(
                ent_hbm.at[idx_v.at[pl.ds(c * CH, CH)]], bufs[b], sems[b])
            if prev is not None:
                pd, pb, pc = prev
                pd.wait()
                pltpu.sync_copy(bufs[pb], out_hbm.at[pl.ds(base + pc * CH, CH)])
            prev = (d, b, c)
        pd, pb, pc = prev
        pd.wait()
        pltpu.sync_copy(bufs[pb], out_hbm.at[pl.ds(base + pc * CH, CH)])

    return sc_gather


def _renorm(e):
    n = jnp.sqrt(jnp.sum(e * e, axis=-1, keepdims=True))
    return e * jnp.where(n > 1.0, 1.0 / (n + 1e-7), 1.0)


def _leaky(x):
    return jnp.where(x >= 0, x, 0.2 * x)


def _tc_body(K, rid_ref, h_ref, u_ref, t_ref, rel_ref,
             W1_ref, b1_ref, W2_ref, b2_ref,
             wl1_ref, wl1b_ref, wl2_ref, wl2b_ref, wl3_ref, wl3b_ref,
             out_ref):
    f32 = jnp.float32
    rel = _renorm(rel_ref[...])
    user = _renorm(u_ref[...])
    h = _renorm(h_ref[...])

    UR = jnp.dot(user, rel.T, preferred_element_type=f32)
    rid = rid_ref[...]
    ur = jnp.zeros(rid.shape, f32)
    for r in range(rel.shape[0]):
        ur = jnp.where(rid == r, UR[:, r:r + 1], ur)

    m = jnp.max(ur, axis=-1, keepdims=True)
    e = jnp.exp(ur - m)
    w = e / jnp.sum(e, axis=-1, keepdims=True)

    s1 = jnp.zeros(h.shape, f32)
    s2 = jnp.zeros(h.shape, f32)
    for k in range(K):
        t = t_ref[:, k, :]
        n = jnp.sqrt(jnp.sum(t * t, axis=-1, keepdims=True))
        t = t * jnp.where(n > 1.0, 1.0 / (n + 1e-7), 1.0)
        wt = w[:, k:k + 1] * t
        s1 = s1 + wt
        s2 = s2 + wt * wt
    Nh = s1 * s1 - s2

    W1 = W1_ref[...]
    W2 = W2_ref[...]
    b1 = b1_ref[...]
    b2 = b2_ref[...]
    item = (_leaky(jnp.dot(h + Nh, W1, preferred_element_type=f32) + b1)
            + _leaky(jnp.dot(h * Nh, W2, preferred_element_type=f32) + b2))
    uo = (_leaky(jnp.dot(user + user, W1, preferred_element_type=f32) + b1)
          + _leaky(jnp.dot(user * user, W2, preferred_element_type=f32) + b2))

    D = h.shape[-1]
    wl1 = wl1_ref[...]
    l1 = (jnp.dot(uo, wl1[0:D], preferred_element_type=f32)
          + jnp.dot(item, wl1[D:2 * D], preferred_element_type=f32)
          + jnp.dot(uo + item, wl1[2 * D:3 * D], preferred_element_type=f32)
          + jnp.dot(uo * item, wl1[3 * D:4 * D], preferred_element_type=f32)
          + wl1b_ref[...])
    l2 = jnp.dot(l1, wl2_ref[...], preferred_element_type=f32) + wl2b_ref[...]
    l3 = jnp.dot(l2, wl3_ref[...], preferred_element_type=f32) + wl3b_ref[...]
    out_ref[...] = 1.0 / (1.0 + jnp.exp(-l3))


def _tc_compute(r_ids, h_rows, u_rows, t3, rel_table,
                W1_w, W1_b, W2_w, W2_b, wl1_w, wl1_b, wl2_w, wl2_b,
                wl3_w, wl3_b):
    B, K = r_ids.shape
    D = h_rows.shape[-1]
    NREL = rel_table.shape[0]
    bb = 512
    grid = (B // bb,)

    def full(shape):
        return pl.BlockSpec(shape, lambda b: (0,) * len(shape))

    out = pl.pallas_call(
        functools.partial(_tc_body, K),
        grid=grid,
        in_specs=[
            pl.BlockSpec((bb, K), lambda b: (b, 0)),
            pl.BlockSpec((bb, D), lambda b: (b, 0)),
            pl.BlockSpec((bb, D), lambda b: (b, 0)),
            pl.BlockSpec((bb, K, D), lambda b: (b, 0, 0)),
            full((NREL, D)),
            full((D, D)), full((D,)),
            full((D, D)), full((D,)),
            full((4 * D, D)), full((D,)),
            full((D, D // 2)), full((D // 2,)),
            full((D // 2, 1)), full((1,)),
        ],
        out_specs=pl.BlockSpec((bb, 1), lambda b: (b, 0)),
        out_shape=jax.ShapeDtypeStruct((B, 1), jnp.float32),
    )(r_ids, h_rows, u_rows, t3, rel_table,
      W1_w, W1_b, W2_w, W2_b, wl1_w, wl1_b, wl2_w, wl2_b, wl3_w, wl3_b)
    return out[:, 0]


def kernel(u, i, adj_entity, adj_relation, entity_table, relation_table,
           W1_w, W1_b, W2_w, W2_b, wl1_w, wl1_b, wl2_w, wl2_b, wl3_w, wl3_b):
    B = u.shape[0]
    N, K = adj_entity.shape
    D = entity_table.shape[1]
    info = plsc.get_sparse_core_info()
    NC, NS = info.num_cores, info.num_subcores

    adj_ec = adj_entity.reshape(N * K // 128, 128)
    adj_rc = adj_relation.reshape(N * K // 128, 128)
    aec, arc, h_rows, u_rows = _make_sc_ids(B, K, D, NC, NS)(
        u, i, adj_ec, adj_rc, entity_table)
    e_ids, r_ids = _tc_extract(i, aec, arc)
    t_rows = _make_sc_gather(B * K, D, NC, NS)(
        e_ids.reshape(B * K), entity_table)
    return _tc_compute(r_ids, h_rows, u_rows, t_rows.reshape(B, K, D),
                       relation_table, W1_w, W1_b, W2_w, W2_b,
                       wl1_w, wl1_b, wl2_w, wl2_b, wl3_w, wl3_b)


# speedup vs baseline: 1.9975x; 1.1058x over previous
import functools

import jax
import jax.numpy as jnp
from jax import lax
from jax.experimental import pallas as pl
from jax.experimental.pallas import tpu as pltpu, tpu_sc as plsc


def _make_sc_all(B, K, D, NC, NS):
    NW = NC * NS
    bw = B // NW
    CH = 128
    nch = bw * K // CH
    mesh = plsc.VectorSubcoreMesh(core_axis_name="c", subcore_axis_name="s")

    @functools.partial(
        pl.kernel,
        mesh=mesh,
        out_type=[
            jax.ShapeDtypeStruct((B, K), jnp.int32),
            jax.ShapeDtypeStruct((B, D), jnp.float32),
            jax.ShapeDtypeStruct((B, D), jnp.float32),
            jax.ShapeDtypeStruct((B * K, D), jnp.float32),
        ],
        scratch_types=[
            pltpu.VMEM((bw,), jnp.int32),
            pltpu.VMEM((bw,), jnp.int32),
            pltpu.VMEM((bw, K), jnp.int32),
            pltpu.VMEM((bw, K), jnp.int32),
            pltpu.VMEM((bw * K,), jnp.int32),
            pltpu.VMEM((bw, D), jnp.float32),
            pltpu.VMEM((bw, D), jnp.float32),
            pltpu.VMEM((CH, D), jnp.float32),
            pltpu.VMEM((CH, D), jnp.float32),
            pltpu.SemaphoreType.DMA,
            pltpu.SemaphoreType.DMA,
            pltpu.SemaphoreType.DMA,
            pltpu.SemaphoreType.DMA,
            pltpu.SemaphoreType.DMA,
            pltpu.SemaphoreType.DMA,
        ],
        compiler_params=pltpu.CompilerParams(use_tc_tiling_on_sc=False),
    )
    def sc_all(u_hbm, i_hbm, adj_e_hbm, adj_r_hbm, ent_hbm,
               rid_out, h_out, u_out, t_out,
               i_v, u_v, eid_v, rid_v, eflat_v, h_v, uu_v, tb0, tb1,
               s0, s1, s2, s3, g0, g1):
        wid = lax.axis_index("s") * NC + lax.axis_index("c")
        base = wid * bw
        pltpu.sync_copy(i_hbm.at[pl.ds(base, bw)], i_v)
        pltpu.sync_copy(u_hbm.at[pl.ds(base, bw)], u_v)
        c0 = pltpu.async_copy(adj_e_hbm.at[i_v], eid_v, s0)
        c1 = pltpu.async_copy(adj_r_hbm.at[i_v], rid_v, s1)
        c2 = pltpu.async_copy(ent_hbm.at[i_v], h_v, s2)
        c3 = pltpu.async_copy(ent_hbm.at[u_v], uu_v, s3)

        c0.wait()
        for b in range(bw):
            eflat_v[pl.ds(b * K, K)] = eid_v[b, :]

        bufs = (tb0, tb1)
        sems = (g0, g1)
        tbase = wid * (bw * K)
        prev = None
        for c in range(nch):
            bsel = c % 2
            d = pltpu.async_copy(
                ent_hbm.at[eflat_v.at[pl.ds(c * CH, CH)]], bufs[bsel],
                sems[bsel])
            if prev is not None:
                pd, pb, pc = prev
                pd.wait()
                pltpu.sync_copy(bufs[pb], t_out.at[pl.ds(tbase + pc * CH, CH)])
            prev = (d, bsel, c)

        c1.wait()
        pltpu.sync_copy(rid_v, rid_out.at[pl.ds(base, bw)])
        c2.wait()
        pltpu.sync_copy(h_v, h_out.at[pl.ds(base, bw)])
        c3.wait()
        pltpu.sync_copy(uu_v, u_out.at[pl.ds(base, bw)])

        pd, pb, pc = prev
        pd.wait()
        pltpu.sync_copy(bufs[pb], t_out.at[pl.ds(tbase + pc * CH, CH)])

    return sc_all


def _renorm(e):
    n = jnp.sqrt(jnp.sum(e * e, axis=-1, keepdims=True))
    return e * jnp.where(n > 1.0, 1.0 / (n + 1e-7), 1.0)


def _leaky(x):
    return jnp.where(x >= 0, x, 0.2 * x)


def _tc_body(K, rid_ref, h_ref, u_ref, t_ref, rel_ref,
             W1_ref, b1_ref, W2_ref, b2_ref,
             wl1_ref, wl1b_ref, wl2_ref, wl2b_ref, wl3_ref, wl3b_ref,
             out_ref):
    f32 = jnp.float32
    rel = _renorm(rel_ref[...])
    user = _renorm(u_ref[...])
    h = _renorm(h_ref[...])

    UR = jnp.dot(user, rel.T, preferred_element_type=f32)
    rid = rid_ref[...]
    ur = jnp.zeros(rid.shape, f32)
    for r in range(rel.shape[0]):
        ur = jnp.where(rid == r, UR[:, r:r + 1], ur)

    m = jnp.max(ur, axis=-1, keepdims=True)
    e = jnp.exp(ur - m)
    w = e / jnp.sum(e, axis=-1, keepdims=True)

    s1 = jnp.zeros(h.shape, f32)
    s2 = jnp.zeros(h.shape, f32)
    for k in range(K):
        t = t_ref[:, k, :]
        n = jnp.sqrt(jnp.sum(t * t, axis=-1, keepdims=True))
        t = t * jnp.where(n > 1.0, 1.0 / (n + 1e-7), 1.0)
        wt = w[:, k:k + 1] * t
        s1 = s1 + wt
        s2 = s2 + wt * wt
    Nh = s1 * s1 - s2

    W1 = W1_ref[...]
    W2 = W2_ref[...]
    b1 = b1_ref[...]
    b2 = b2_ref[...]
    item = (_leaky(jnp.dot(h + Nh, W1, preferred_element_type=f32) + b1)
            + _leaky(jnp.dot(h * Nh, W2, preferred_element_type=f32) + b2))
    uo = (_leaky(jnp.dot(user + user, W1, preferred_element_type=f32) + b1)
          + _leaky(jnp.dot(user * user, W2, preferred_element_type=f32) + b2))

    D = h.shape[-1]
    wl1 = wl1_ref[...]
    l1 = (jnp.dot(uo, wl1[0:D], preferred_element_type=f32)
          + jnp.dot(item, wl1[D:2 * D], preferred_element_type=f32)
          + jnp.dot(uo + item, wl1[2 * D:3 * D], preferred_element_type=f32)
          + jnp.dot(uo * item, wl1[3 * D:4 * D], preferred_element_type=f32)
          + wl1b_ref[...])
    l2 = jnp.dot(l1, wl2_ref[...], preferred_element_type=f32) + wl2b_ref[...]
    l3 = jnp.dot(l2, wl3_ref[...], preferred_element_type=f32) + wl3b_ref[...]
    out_ref[...] = 1.0 / (1.0 + jnp.exp(-l3))


def _tc_compute(r_ids, h_rows, u_rows, t3, rel_table,
                W1_w, W1_b, W2_w, W2_b, wl1_w, wl1_b, wl2_w, wl2_b,
                wl3_w, wl3_b):
    B, K = r_ids.shape
    D = h_rows.shape[-1]
    NREL = rel_table.shape[0]
    bb = 512
    grid = (B // bb,)

    def full(shape):
        return pl.BlockSpec(shape, lambda b: (0,) * len(shape))

    out = pl.pallas_call(
        functools.partial(_tc_body, K),
        grid=grid,
        in_specs=[
            pl.BlockSpec((bb, K), lambda b: (b, 0)),
            pl.BlockSpec((bb, D), lambda b: (b, 0)),
            pl.BlockSpec((bb, D), lambda b: (b, 0)),
            pl.BlockSpec((bb, K, D), lambda b: (b, 0, 0)),
            full((NREL, D)),
            full((D, D)), full((D,)),
            full((D, D)), full((D,)),
            full((4 * D, D)), full((D,)),
            full((D, D // 2)), full((D // 2,)),
            full((D // 2, 1)), full((1,)),
        ],
        out_specs=pl.BlockSpec((bb, 1), lambda b: (b, 0)),
        out_shape=jax.ShapeDtypeStruct((B, 1), jnp.float32),
    )(r_ids, h_rows, u_rows, t3, rel_table,
      W1_w, W1_b, W2_w, W2_b, wl1_w, wl1_b, wl2_w, wl2_b, wl3_w, wl3_b)
    return out[:, 0]


def kernel(u, i, adj_entity, adj_relation, entity_table, relation_table,
           W1_w, W1_b, W2_w, W2_b, wl1_w, wl1_b, wl2_w, wl2_b, wl3_w, wl3_b):
    B = u.shape[0]
    N, K = adj_entity.shape
    D = entity_table.shape[1]
    info = plsc.get_sparse_core_info()
    NC, NS = info.num_cores, info.num_subcores

    r_ids, h_rows, u_rows, t_rows = _make_sc_all(B, K, D, NC, NS)(
        u, i, adj_entity, adj_relation, entity_table)
    return _tc_compute(r_ids, h_rows, u_rows, t_rows.reshape(B, K, D),
                       relation_table, W1_w, W1_b, W2_w, W2_b,
                       wl1_w, wl1_b, wl2_w, wl2_b, wl3_w, wl3_b)


# speedup vs baseline: 2.5898x; 1.2965x over previous
import functools

import jax
import jax.numpy as jnp
from jax import lax
from jax.experimental import pallas as pl
from jax.experimental.pallas import tpu as pltpu, tpu_sc as plsc


def _make_sc_all(B, K, D, NC, NS):
    NW = NC * NS
    bw = B // NW
    CH = 128
    nch = bw * K // CH
    mesh = plsc.VectorSubcoreMesh(core_axis_name="c", subcore_axis_name="s")

    @functools.partial(
        pl.kernel,
        mesh=mesh,
        out_type=[
            jax.ShapeDtypeStruct((B, K), jnp.int32),
            jax.ShapeDtypeStruct((B, D), jnp.float32),
            jax.ShapeDtypeStruct((B, D), jnp.float32),
            jax.ShapeDtypeStruct((B * K, D), jnp.float32),
        ],
        scratch_types=[
            pltpu.VMEM((bw,), jnp.int32),
            pltpu.VMEM((bw,), jnp.int32),
            pltpu.VMEM((bw, K), jnp.int32),
            pltpu.VMEM((bw, K), jnp.int32),
            pltpu.VMEM((bw * K,), jnp.int32),
            pltpu.VMEM((bw, D), jnp.float32),
            pltpu.VMEM((bw, D), jnp.float32),
            pltpu.VMEM((CH, D), jnp.float32),
            pltpu.VMEM((CH, D), jnp.float32),
            pltpu.SemaphoreType.DMA,
            pltpu.SemaphoreType.DMA,
            pltpu.SemaphoreType.DMA,
            pltpu.SemaphoreType.DMA,
            pltpu.SemaphoreType.DMA,
            pltpu.SemaphoreType.DMA,
        ],
        compiler_params=pltpu.CompilerParams(use_tc_tiling_on_sc=False),
    )
    def sc_all(u_hbm, i_hbm, adj_e_hbm, adj_r_hbm, ent_hbm,
               rid_out, h_out, u_out, t_out,
               i_v, u_v, eid_v, rid_v, eflat_v, h_v, uu_v, tb0, tb1,
               s0, s1, s2, s3, g0, g1):
        wid = lax.axis_index("s") * NC + lax.axis_index("c")
        base = wid * bw
        pltpu.sync_copy(i_hbm.at[pl.ds(base, bw)], i_v)
        pltpu.sync_copy(u_hbm.at[pl.ds(base, bw)], u_v)
        c0 = pltpu.async_copy(adj_e_hbm.at[i_v], eid_v, s0)
        c1 = pltpu.async_copy(adj_r_hbm.at[i_v], rid_v, s1)
        c2 = pltpu.async_copy(ent_hbm.at[i_v], h_v, s2)
        c3 = pltpu.async_copy(ent_hbm.at[u_v], uu_v, s3)

        c0.wait()
        for b in range(bw):
            eflat_v[pl.ds(b * K, K)] = eid_v[b, :]

        bufs = (tb0, tb1)
        sems = (g0, g1)
        tbase = wid * (bw * K)
        prev = None
        for c in range(nch):
            bsel = c % 2
            d = pltpu.async_copy(
                ent_hbm.at[eflat_v.at[pl.ds(c * CH, CH)]], bufs[bsel],
                sems[bsel])
            if prev is not None:
                pd, pb, pc = prev
                pd.wait()
                pltpu.sync_copy(bufs[pb], t_out.at[pl.ds(tbase + pc * CH, CH)])
            prev = (d, bsel, c)

        c1.wait()
        pltpu.sync_copy(rid_v, rid_out.at[pl.ds(base, bw)])
        c2.wait()
        pltpu.sync_copy(h_v, h_out.at[pl.ds(base, bw)])
        c3.wait()
        pltpu.sync_copy(uu_v, u_out.at[pl.ds(base, bw)])

        pd, pb, pc = prev
        pd.wait()
        pltpu.sync_copy(bufs[pb], t_out.at[pl.ds(tbase + pc * CH, CH)])

    return sc_all


def _renorm(e):
    n2 = jnp.sum(e * e, axis=-1, keepdims=True)
    return e * jnp.where(n2 > 1.0, lax.rsqrt(n2), 1.0)


def _leaky(x):
    return jnp.where(x >= 0, x, 0.2 * x)


def _tc_body(K, rid_ref, h_ref, u_ref, t_ref, rel_ref,
             W1_ref, b1_ref, W2_ref, b2_ref,
             wl1_ref, wl1b_ref, wl2_ref, wl2b_ref, wl3_ref, wl3b_ref,
             out_ref):
    f32 = jnp.float32
    rel = _renorm(rel_ref[...])
    user = _renorm(u_ref[...])
    h = _renorm(h_ref[...])

    UR = jnp.dot(user, rel.T, preferred_element_type=f32)
    rid = rid_ref[...]
    ur = jnp.zeros(rid.shape, f32)
    for r in range(rel.shape[0]):
        ur = jnp.where(rid == r, UR[:, r:r + 1], ur)

    m = jnp.max(ur, axis=-1, keepdims=True)
    e = jnp.exp(ur - m)
    w = e / jnp.sum(e, axis=-1, keepdims=True)

    t = t_ref[...]
    n2 = jnp.sum(t * t, axis=2, keepdims=True)
    scale = w[:, :, None] * jnp.where(n2 > 1.0, lax.rsqrt(n2), 1.0)
    wt = scale * t
    s1 = jnp.sum(wt, axis=1)
    s2 = jnp.sum(wt * wt, axis=1)
    Nh = s1 * s1 - s2

    W1 = W1_ref[...]
    W2 = W2_ref[...]
    b1 = b1_ref[...]
    b2 = b2_ref[...]
    item = (_leaky(jnp.dot(h + Nh, W1, preferred_element_type=f32) + b1)
            + _leaky(jnp.dot(h * Nh, W2, preferred_element_type=f32) + b2))
    uo = (_leaky(jnp.dot(user + user, W1, preferred_element_type=f32) + b1)
          + _leaky(jnp.dot(user * user, W2, preferred_element_type=f32) + b2))

    D = h.shape[-1]
    wl1 = wl1_ref[...]
    l1 = (jnp.dot(uo, wl1[0:D], preferred_element_type=f32)
          + jnp.dot(item, wl1[D:2 * D], preferred_element_type=f32)
          + jnp.dot(uo + item, wl1[2 * D:3 * D], preferred_element_type=f32)
          + jnp.dot(uo * item, wl1[3 * D:4 * D], preferred_element_type=f32)
          + wl1b_ref[...])
    l2 = jnp.dot(l1, wl2_ref[...], preferred_element_type=f32) + wl2b_ref[...]
    l3 = jnp.dot(l2, wl3_ref[...], preferred_element_type=f32) + wl3b_ref[...]
    out_ref[...] = 1.0 / (1.0 + jnp.exp(-l3))


def _tc_compute(r_ids, h_rows, u_rows, t3, rel_table,
                W1_w, W1_b, W2_w, W2_b, wl1_w, wl1_b, wl2_w, wl2_b,
                wl3_w, wl3_b):
    B, K = r_ids.shape
    D = h_rows.shape[-1]
    NREL = rel_table.shape[0]
    bb = 512
    grid = (B // bb,)

    def full(shape):
        return pl.BlockSpec(shape, lambda b: (0,) * len(shape))

    out = pl.pallas_call(
        functools.partial(_tc_body, K),
        grid=grid,
        in_specs=[
            pl.BlockSpec((bb, K), lambda b: (b, 0)),
            pl.BlockSpec((bb, D), lambda b: (b, 0)),
            pl.BlockSpec((bb, D), lambda b: (b, 0)),
            pl.BlockSpec((bb, K, D), lambda b: (b, 0, 0)),
            full((NREL, D)),
            full((D, D)), full((D,)),
            full((D, D)), full((D,)),
            full((4 * D, D)), full((D,)),
            full((D, D // 2)), full((D // 2,)),
            full((D // 2, 1)), full((1,)),
        ],
        out_specs=pl.BlockSpec((bb, 1), lambda b: (b, 0)),
        out_shape=jax.ShapeDtypeStruct((B, 1), jnp.float32),
    )(r_ids, h_rows, u_rows, t3, rel_table,
      W1_w, W1_b, W2_w, W2_b, wl1_w, wl1_b, wl2_w, wl2_b, wl3_w, wl3_b)
    return out[:, 0]


def kernel(u, i, adj_entity, adj_relation, entity_table, relation_table,
           W1_w, W1_b, W2_w, W2_b, wl1_w, wl1_b, wl2_w, wl2_b, wl3_w, wl3_b):
    B = u.shape[0]
    N, K = adj_entity.shape
    D = entity_table.shape[1]
    info = plsc.get_sparse_core_info()
    NC, NS = info.num_cores, info.num_subcores

    r_ids, h_rows, u_rows, t_rows = _make_sc_all(B, K, D, NC, NS)(
        u, i, adj_entity, adj_relation, entity_table)
    return _tc_compute(r_ids, h_rows, u_rows, t_rows.reshape(B, K, D),
                       relation_table, W1_w, W1_b, W2_w, W2_b,
                       wl1_w, wl1_b, wl2_w, wl2_b, wl3_w, wl3_b)
